# baseline (device time: 59379 ns/iter reference)
import jax
import jax.numpy as jnp
from jax import lax
from jax.experimental import pallas as pl
from jax.experimental.pallas import tpu as pltpu

N_DEV = 4
K_DMA = 32


def _body(pos_ref, idx_ref, cnt_ref, e_ref, out_ref,
          bx1, bx2, by1, by2, gather_sems, xs, xr, ys, yr):
    t, d = out_ref.shape
    t2 = t // 2
    b2 = t // 4
    b4 = t // 8

    my = lax.axis_index("i")
    a = my % 2
    b = my // 2
    k1 = (a + b) % 2
    p_a = my + 1 - 2 * a
    p_b = 3 - my

    def xfer(src_rows, n_rows, dst, ssem, rsem, peer):
        return pltpu.make_async_remote_copy(
            src_ref=out_ref.at[pl.ds(src_rows, n_rows), :],
            dst_ref=dst,
            send_sem=ssem,
            recv_sem=rsem,
            device_id=(peer,),
            device_id_type=pl.DeviceIdType.MESH,
        )

    def row_dma(j):
        return pltpu.make_async_copy(
            e_ref.at[pl.ds(idx_ref[j], 1), :],
            out_ref.at[pl.ds(pos_ref[j], 1), :],
            gather_sems.at[j % K_DMA],
        )

    def gather(lo, hi):
        def step(j, carry):
            @pl.when(j - lo >= K_DMA)
            def _():
                row_dma(j - K_DMA).wait()
            row_dma(j).start()
            return carry

        lax.fori_loop(lo, hi, step, 0)

        def drain(j, carry):
            row_dma(j).wait()
            return carry

        lax.fori_loop(jnp.maximum(hi - K_DMA, lo), hi, drain, 0)

    barrier_sem = pltpu.get_barrier_semaphore()
    for nbr in [p_a, p_b]:
        pl.semaphore_signal(
            barrier_sem, inc=1,
            device_id=(nbr,), device_id_type=pl.DeviceIdType.MESH,
        )
    pl.semaphore_wait(barrier_sem, 2)

    out_ref[:, :] = jnp.zeros((t, d), jnp.float32)

    nx = cnt_ref[0]
    ny = cnt_ref[1]
    n = cnt_ref[2]

    x_send = (1 - k1) * b2
    y_send = t2 + (1 - b) * b2
    gather(0, nx)
    x1 = xfer(x_send, b2, bx1, xs.at[0], xr.at[0], p_a)
    x1.start()
    gather(nx, ny)
    y1 = xfer(y_send, b2, by1, ys.at[0], yr.at[0], p_b)
    y1.start()

    gather(ny, n)

    x_keep = k1 * b2
    y_keep = t2 + b * b2
    x_q_keep = x_keep + b * b4
    y_q_keep = y_keep + a * b4

    def add_quarter(base, q, buf):
        out_ref[pl.ds(base + q, b4), :] = (
            out_ref[pl.ds(base + q, b4), :] + buf[pl.ds(q, b4), :]
        )

    x1.wait()
    add_quarter(x_keep, (1 - b) * b4, bx1)
    x2 = xfer(x_keep + (1 - b) * b4, b4, bx2, xs.at[1], xr.at[1], p_b)
    x2.start()
    add_quarter(x_keep, b * b4, bx1)
    y1.wait()
    add_quarter(y_keep, (1 - a) * b4, by1)
    y2 = xfer(y_keep + (1 - a) * b4, b4, by2, ys.at[1], yr.at[1], p_a)
    y2.start()
    add_quarter(y_keep, a * b4, by1)

    x2.wait()
    out_ref[pl.ds(x_q_keep, b4), :] = (
        out_ref[pl.ds(x_q_keep, b4), :] + bx2[:, :]
    )
    x3 = xfer(x_q_keep, b4, out_ref.at[pl.ds(x_q_keep, b4), :],
              xs.at[2], xr.at[2], p_b)
    x3.start()
    y2.wait()
    out_ref[pl.ds(y_q_keep, b4), :] = (
        out_ref[pl.ds(y_q_keep, b4), :] + by2[:, :]
    )
    y3 = xfer(y_q_keep, b4, out_ref.at[pl.ds(y_q_keep, b4), :],
              ys.at[2], yr.at[2], p_a)
    y3.start()

    x3.wait()
    x4 = xfer(x_keep, b2, out_ref.at[pl.ds(x_keep, b2), :],
              xs.at[3], xr.at[3], p_a)
    x4.start()
    y3.wait()
    y4 = xfer(y_keep, b2, out_ref.at[pl.ds(y_keep, b2), :],
              ys.at[3], yr.at[3], p_b)
    y4.start()
    x4.wait()
    y4.wait()


def kernel(ids, E):
    v_per, d = E.shape
    t = ids.shape[0]
    my_pos = lax.axis_index("i")

    local = ids - my_pos * v_per
    mask = (local >= 0) & (local < v_per)

    a = my_pos % 2
    b = my_pos // 2
    k1 = (a + b) % 2
    b2 = t // 4
    x_send = (1 - k1) * b2
    y_send = t // 2 + (1 - b) * b2
    rows = jnp.arange(t, dtype=jnp.int32)
    in_xs = (rows >= x_send) & (rows < x_send + b2)
    in_ys = (rows >= y_send) & (rows < y_send + b2)

    csx = jnp.cumsum((mask & in_xs).astype(jnp.int32))
    csy = jnp.cumsum((mask & in_ys).astype(jnp.int32))
    ct = jnp.cumsum(mask.astype(jnp.int32))
    nx = csx[-1]
    ny = nx + csy[-1]
    slot = jnp.where(
        in_xs, csx, jnp.where(in_ys, nx + csy, ny + ct - csx - csy)
    ) - 1
    slots = jnp.arange(t, dtype=jnp.int32)
    m = ((slot[:, None] == slots[None, :]) & mask[:, None]).astype(jnp.int32)
    lidx = jnp.clip(local, 0, v_per - 1)
    packed = jnp.sum(m * (lidx[:, None] * t + rows[:, None]), axis=0)
    pos = (packed % t).astype(jnp.int32)
    idx = (packed // t).astype(jnp.int32)
    cnt = jnp.stack([nx, ny, ct[-1]]).astype(jnp.int32)

    return pl.pallas_call(
        _body,
        out_shape=jax.ShapeDtypeStruct((t, d), jnp.float32),
        in_specs=[
            pl.BlockSpec(memory_space=pltpu.SMEM),
            pl.BlockSpec(memory_space=pltpu.SMEM),
            pl.BlockSpec(memory_space=pltpu.SMEM),
            pl.BlockSpec(memory_space=pltpu.MemorySpace.HBM),
        ],
        out_specs=pl.BlockSpec(memory_space=pltpu.VMEM),
        scratch_shapes=[
            pltpu.VMEM((t // 4, d), jnp.float32),
            pltpu.VMEM((t // 8, d), jnp.float32),
            pltpu.VMEM((t // 4, d), jnp.float32),
            pltpu.VMEM((t // 8, d), jnp.float32),
            pltpu.SemaphoreType.DMA((K_DMA,)),
            pltpu.SemaphoreType.DMA((4,)),
            pltpu.SemaphoreType.DMA((4,)),
            pltpu.SemaphoreType.DMA((4,)),
            pltpu.SemaphoreType.DMA((4,)),
        ],
        compiler_params=pltpu.CompilerParams(collective_id=0),
    )(pos, idx, cnt, E)


# device time: 58419 ns/iter; 1.0164x vs baseline; 1.0164x over previous
import jax
import jax.numpy as jnp
from jax import lax
from jax.experimental import pallas as pl
from jax.experimental.pallas import tpu as pltpu

N_DEV = 4
K_DMA = 32


def _body(pos_ref, idx_ref, cnt_ref, e_ref, out_ref,
          bx1, bx2, by1, by2, gather_sems, xs, xr, ys, yr):
    t, d = out_ref.shape
    t2 = t // 2
    b2 = t // 4
    b4 = t // 8

    my = lax.axis_index("i")
    a = my % 2
    b = my // 2
    k1 = (a + b) % 2
    p_a = my + 1 - 2 * a
    p_b = 3 - my

    def xfer(src_rows, n_rows, dst, ssem, rsem, peer):
        return pltpu.make_async_remote_copy(
            src_ref=out_ref.at[pl.ds(src_rows, n_rows), :],
            dst_ref=dst,
            send_sem=ssem,
            recv_sem=rsem,
            device_id=(peer,),
            device_id_type=pl.DeviceIdType.MESH,
        )

    def row_dma(j):
        return pltpu.make_async_copy(
            e_ref.at[pl.ds(idx_ref[j], 1), :],
            out_ref.at[pl.ds(pos_ref[j], 1), :],
            gather_sems.at[j % K_DMA],
        )

    def gather(lo, hi):
        def step(j, carry):
            @pl.when(j - lo >= K_DMA)
            def _():
                row_dma(j - K_DMA).wait()
            row_dma(j).start()
            return carry

        lax.fori_loop(lo, hi, step, 0)

        def drain(j, carry):
            row_dma(j).wait()
            return carry

        lax.fori_loop(jnp.maximum(hi - K_DMA, lo), hi, drain, 0)

    barrier_sem = pltpu.get_barrier_semaphore()
    for nbr in [p_a, p_b]:
        pl.semaphore_signal(
            barrier_sem, inc=1,
            device_id=(nbr,), device_id_type=pl.DeviceIdType.MESH,
        )
    pl.semaphore_wait(barrier_sem, 2)

    out_ref[:, :] = jnp.zeros((t, d), jnp.float32)

    nx = cnt_ref[0]
    ny = cnt_ref[1]
    n = cnt_ref[2]

    x_send = (1 - k1) * b2
    y_send = t2 + (1 - b) * b2
    gather(0, nx)
    x1 = xfer(x_send, b2, bx1, xs.at[0], xr.at[0], p_a)
    x1.start()
    gather(nx, ny)
    y1 = xfer(y_send, b2, by1, ys.at[0], yr.at[0], p_b)
    y1.start()

    gather(ny, n)

    x_keep = k1 * b2
    y_keep = t2 + b * b2
    x_q_keep = x_keep + b * b4
    y_q_keep = y_keep + a * b4

    def add_quarter(base, q, buf):
        out_ref[pl.ds(base + q, b4), :] = (
            out_ref[pl.ds(base + q, b4), :] + buf[pl.ds(q, b4), :]
        )

    x1.wait()
    add_quarter(x_keep, (1 - b) * b4, bx1)
    x2 = xfer(x_keep + (1 - b) * b4, b4, bx2, xs.at[1], xr.at[1], p_b)
    x2.start()
    add_quarter(x_keep, b * b4, bx1)
    y1.wait()
    add_quarter(y_keep, (1 - a) * b4, by1)
    y2 = xfer(y_keep + (1 - a) * b4, b4, by2, ys.at[1], yr.at[1], p_a)
    y2.start()
    add_quarter(y_keep, a * b4, by1)

    x2.wait()
    out_ref[pl.ds(x_q_keep, b4), :] = (
        out_ref[pl.ds(x_q_keep, b4), :] + bx2[:, :]
    )
    x3 = xfer(x_q_keep, b4, out_ref.at[pl.ds(x_q_keep, b4), :],
              xs.at[2], xr.at[2], p_b)
    x3.start()
    y2.wait()
    out_ref[pl.ds(y_q_keep, b4), :] = (
        out_ref[pl.ds(y_q_keep, b4), :] + by2[:, :]
    )
    y3 = xfer(y_q_keep, b4, out_ref.at[pl.ds(y_q_keep, b4), :],
              ys.at[2], yr.at[2], p_a)
    y3.start()

    x3.wait()
    x4 = xfer(x_keep, b2, out_ref.at[pl.ds(x_keep, b2), :],
              xs.at[3], xr.at[3], p_a)
    x4.start()
    y3.wait()
    y4 = xfer(y_keep, b2, out_ref.at[pl.ds(y_keep, b2), :],
              ys.at[3], yr.at[3], p_b)
    y4.start()
    x4.wait()
    y4.wait()


def kernel(ids, E):
    v_per, d = E.shape
    t = ids.shape[0]
    my_pos = lax.axis_index("i")

    local = ids - my_pos * v_per
    mask = (local >= 0) & (local < v_per)

    a = my_pos % 2
    b = my_pos // 2
    k1 = (a + b) % 2
    b2 = t // 4
    x_send = (1 - k1) * b2
    y_send = t // 2 + (1 - b) * b2
    rows = jnp.arange(t, dtype=jnp.int32)
    in_xs = (rows >= x_send) & (rows < x_send + b2)
    in_ys = (rows >= y_send) & (rows < y_send + b2)

    packed_xy = jnp.cumsum(
        (mask & in_xs).astype(jnp.int32)
        + 2048 * (mask & in_ys).astype(jnp.int32)
    )
    csx = packed_xy % 2048
    csy = packed_xy // 2048
    ct = jnp.cumsum(mask.astype(jnp.int32))
    nx = csx[-1]
    ny = nx + csy[-1]
    slot = jnp.where(
        in_xs, csx, jnp.where(in_ys, nx + csy, ny + ct - csx - csy)
    ) - 1
    slots = jnp.arange(t, dtype=jnp.int32)
    m = ((slot[:, None] == slots[None, :]) & mask[:, None]).astype(jnp.int32)
    pos = jnp.sum(m * rows[:, None], axis=0).astype(jnp.int32)
    idx = jnp.clip(jnp.sum(m * local[:, None], axis=0),
                   0, v_per - 1).astype(jnp.int32)
    cnt = jnp.stack([nx, ny, ct[-1]]).astype(jnp.int32)

    return pl.pallas_call(
        _body,
        out_shape=jax.ShapeDtypeStruct((t, d), jnp.float32),
        in_specs=[
            pl.BlockSpec(memory_space=pltpu.SMEM),
            pl.BlockSpec(memory_space=pltpu.SMEM),
            pl.BlockSpec(memory_space=pltpu.SMEM),
            pl.BlockSpec(memory_space=pltpu.MemorySpace.HBM),
        ],
        out_specs=pl.BlockSpec(memory_space=pltpu.VMEM),
        scratch_shapes=[
            pltpu.VMEM((t // 4, d), jnp.float32),
            pltpu.VMEM((t // 8, d), jnp.float32),
            pltpu.VMEM((t // 4, d), jnp.float32),
            pltpu.VMEM((t // 8, d), jnp.float32),
            pltpu.SemaphoreType.DMA((K_DMA,)),
            pltpu.SemaphoreType.DMA((4,)),
            pltpu.SemaphoreType.DMA((4,)),
            pltpu.SemaphoreType.DMA((4,)),
            pltpu.SemaphoreType.DMA((4,)),
        ],
        compiler_params=pltpu.CompilerParams(collective_id=0),
    )(pos, idx, cnt, E)


# device time: 57818 ns/iter; 1.0270x vs baseline; 1.0104x over previous
import os

import jax
import jax.numpy as jnp
from jax import lax
from jax.experimental import pallas as pl
from jax.experimental.pallas import tpu as pltpu

N_DEV = 4
K_DMA = 32

DO_ZERO = os.environ.get("KZ", "1") == "1"
DO_GATHER = os.environ.get("KG", "1") == "1"
DO_COMM = os.environ.get("KC", "1") == "1"


def _body(pos_ref, idx_ref, cnt_ref, e_ref, out_ref,
          bx1, bx2, by1, by2, gather_sems, xs, xr, ys, yr):
    t, d = out_ref.shape
    t2 = t // 2
    b2 = t // 4
    b4 = t // 8

    my = lax.axis_index("i")
    a = my % 2
    b = my // 2
    k1 = (a + b) % 2
    p_a = my + 1 - 2 * a
    p_b = 3 - my

    def xfer(src_rows, n_rows, dst, ssem, rsem, peer):
        return pltpu.make_async_remote_copy(
            src_ref=out_ref.at[pl.ds(src_rows, n_rows), :],
            dst_ref=dst,
            send_sem=ssem,
            recv_sem=rsem,
            device_id=(peer,),
            device_id_type=pl.DeviceIdType.MESH,
        )

    def row_dma(j):
        return pltpu.make_async_copy(
            e_ref.at[pl.ds(idx_ref[j], 1), :],
            out_ref.at[pl.ds(pos_ref[j], 1), :],
            gather_sems.at[j % K_DMA],
        )

    def gather(lo, hi):
        def step(j, carry):
            @pl.when(j - lo >= K_DMA)
            def _():
                row_dma(j - K_DMA).wait()
            row_dma(j).start()
            return carry

        lax.fori_loop(lo, hi, step, 0)

        def drain(j, carry):
            row_dma(j).wait()
            return carry

        lax.fori_loop(jnp.maximum(hi - K_DMA, lo), hi, drain, 0)

    barrier_sem = pltpu.get_barrier_semaphore()
    for nbr in [p_a, p_b]:
        pl.semaphore_signal(
            barrier_sem, inc=1,
            device_id=(nbr,), device_id_type=pl.DeviceIdType.MESH,
        )
    pl.semaphore_wait(barrier_sem, 2)

    nx = cnt_ref[0]
    ny = cnt_ref[1]
    n = cnt_ref[2]

    x_send = (1 - k1) * b2
    y_send = t2 + (1 - b) * b2
    x_keep = k1 * b2
    y_keep = t2 + b * b2
    x_q_keep = x_keep + b * b4
    y_q_keep = y_keep + a * b4

    def zero_half(off):
        out_ref[pl.ds(off, b2), :] = jnp.zeros((b2, d), jnp.float32)

    def add_quarter(base, q, buf):
        out_ref[pl.ds(base + q, b4), :] = (
            out_ref[pl.ds(base + q, b4), :] + buf[pl.ds(q, b4), :]
        )

    if DO_ZERO:
        zero_half(x_send)
    if DO_GATHER:
        gather(0, nx)
    if not DO_COMM:
        if DO_ZERO:
            zero_half(y_send)
            zero_half(x_keep)
            zero_half(y_keep)
        if DO_GATHER:
            gather(nx, n)
        return
    x1a = xfer(x_send + (1 - b) * b4, b4,
               bx1.at[pl.ds((1 - b) * b4, b4), :], xs.at[0], xr.at[0], p_a)
    x1b = xfer(x_send + b * b4, b4,
               bx1.at[pl.ds(b * b4, b4), :], xs.at[1], xr.at[1], p_a)
    x1a.start()
    x1b.start()
    if DO_ZERO:
        zero_half(y_send)
    if DO_GATHER:
        gather(nx, ny)
    y1a = xfer(y_send + a * b4, b4,
               by1.at[pl.ds(a * b4, b4), :], ys.at[0], yr.at[0], p_b)
    y1b = xfer(y_send + (1 - a) * b4, b4,
               by1.at[pl.ds((1 - a) * b4, b4), :], ys.at[1], yr.at[1], p_b)
    y1a.start()
    y1b.start()

    if DO_ZERO:
        zero_half(x_keep)
        zero_half(y_keep)
    if DO_GATHER:
        gather(ny, n)

    x1a.wait()
    add_quarter(x_keep, (1 - b) * b4, bx1)
    x2 = xfer(x_keep + (1 - b) * b4, b4, bx2, xs.at[2], xr.at[2], p_b)
    x2.start()
    y1a.wait()
    add_quarter(y_keep, (1 - a) * b4, by1)
    y2 = xfer(y_keep + (1 - a) * b4, b4, by2, ys.at[2], yr.at[2], p_a)
    y2.start()
    x1b.wait()
    add_quarter(x_keep, b * b4, bx1)
    y1b.wait()
    add_quarter(y_keep, a * b4, by1)

    x2.wait()
    out_ref[pl.ds(x_q_keep, b4), :] = (
        out_ref[pl.ds(x_q_keep, b4), :] + bx2[:, :]
    )
    x3 = xfer(x_q_keep, b4, out_ref.at[pl.ds(x_q_keep, b4), :],
              xs.at[3], xr.at[3], p_b)
    x3.start()
    x4a = xfer(x_q_keep, b4, out_ref.at[pl.ds(x_q_keep, b4), :],
               xs.at[4], xr.at[4], p_a)
    x4a.start()
    y2.wait()
    out_ref[pl.ds(y_q_keep, b4), :] = (
        out_ref[pl.ds(y_q_keep, b4), :] + by2[:, :]
    )
    y3 = xfer(y_q_keep, b4, out_ref.at[pl.ds(y_q_keep, b4), :],
              ys.at[3], yr.at[3], p_a)
    y3.start()
    y4a = xfer(y_q_keep, b4, out_ref.at[pl.ds(y_q_keep, b4), :],
               ys.at[4], yr.at[4], p_b)
    y4a.start()

    x3.wait()
    x4b = xfer(x_keep + (1 - b) * b4, b4,
               out_ref.at[pl.ds(x_keep + (1 - b) * b4, b4), :],
               xs.at[5], xr.at[5], p_a)
    x4b.start()
    y3.wait()
    y4b = xfer(y_keep + (1 - a) * b4, b4,
               out_ref.at[pl.ds(y_keep + (1 - a) * b4, b4), :],
               ys.at[5], yr.at[5], p_b)
    y4b.start()
    x4a.wait()
    x4b.wait()
    y4a.wait()
    y4b.wait()


def kernel(ids, E):
    v_per, d = E.shape
    t = ids.shape[0]
    my_pos = lax.axis_index("i")

    local = ids - my_pos * v_per
    mask = (local >= 0) & (local < v_per)

    a = my_pos % 2
    b = my_pos // 2
    k1 = (a + b) % 2
    b2 = t // 4
    x_send = (1 - k1) * b2
    y_send = t // 2 + (1 - b) * b2
    rows = jnp.arange(t, dtype=jnp.int32)
    in_xs = (rows >= x_send) & (rows < x_send + b2)
    in_ys = (rows >= y_send) & (rows < y_send + b2)

    packed_xy = jnp.cumsum(
        (mask & in_xs).astype(jnp.int32)
        + 2048 * (mask & in_ys).astype(jnp.int32)
    )
    csx = packed_xy % 2048
    csy = packed_xy // 2048
    ct = jnp.cumsum(mask.astype(jnp.int32))
    nx = csx[-1]
    ny = nx + csy[-1]
    slot = jnp.where(
        in_xs, csx, jnp.where(in_ys, nx + csy, ny + ct - csx - csy)
    ) - 1
    slots = jnp.arange(t, dtype=jnp.int32)
    m = (slot[:, None] == slots[None, :]) & mask[:, None]
    pos = jnp.sum(jnp.where(m, rows[:, None], 0), axis=0).astype(jnp.int32)
    idx = jnp.clip(jnp.sum(jnp.where(m, local[:, None], 0), axis=0),
                   0, v_per - 1).astype(jnp.int32)
    cnt = jnp.stack([nx, ny, ct[-1]]).astype(jnp.int32)

    return pl.pallas_call(
        _body,
        out_shape=jax.ShapeDtypeStruct((t, d), jnp.float32),
        in_specs=[
            pl.BlockSpec(memory_space=pltpu.SMEM),
            pl.BlockSpec(memory_space=pltpu.SMEM),
            pl.BlockSpec(memory_space=pltpu.SMEM),
            pl.BlockSpec(memory_space=pltpu.MemorySpace.HBM),
        ],
        out_specs=pl.BlockSpec(memory_space=pltpu.VMEM),
        scratch_shapes=[
            pltpu.VMEM((t // 4, d), jnp.float32),
            pltpu.VMEM((t // 8, d), jnp.float32),
            pltpu.VMEM((t // 4, d), jnp.float32),
            pltpu.VMEM((t // 8, d), jnp.float32),
            pltpu.SemaphoreType.DMA((K_DMA,)),
            pltpu.SemaphoreType.DMA((6,)),
            pltpu.SemaphoreType.DMA((6,)),
            pltpu.SemaphoreType.DMA((6,)),
            pltpu.SemaphoreType.DMA((6,)),
        ],
        compiler_params=pltpu.CompilerParams(collective_id=0),
    )(pos, idx, cnt, E)
